# baseline (speedup 1.0000x reference)
import functools

import jax
import jax.numpy as jnp
import numpy as np
from jax import lax
from jax.experimental import pallas as pl
from jax.experimental.pallas import tpu as pltpu
from jax.experimental.pallas import tpu_sc as plsc

NC, NS, LANES = 2, 16, 16
TILES = NC * NS
K = 80


def _edge_proj_body(ea_ref, wa_ref, wb_ref, ba_ref, bb_ref, o_ref):
    a = (jnp.dot(ea_ref[...], wa_ref[...], preferred_element_type=jnp.float32)
         + ba_ref[...])
    b = (jnp.dot(ea_ref[...], wb_ref[...], preferred_element_type=jnp.float32)
         + bb_ref[...])
    au = jax.lax.bitcast_convert_type(a, jnp.uint32) >> 16
    bu = jax.lax.bitcast_convert_type(b, jnp.uint32) & jnp.uint32(0xFFFF0000)
    o_ref[...] = jax.lax.bitcast_convert_type(au | bu, jnp.int32)


def _edge_proj(ea2, wa2, wb2, ba_row, bb_row):
    Ep2, DE2 = ea2.shape
    D = wa2.shape[1]
    BE = 5 * TILES * K // 2
    assert Ep2 % BE == 0
    return pl.pallas_call(
        _edge_proj_body,
        grid=(Ep2 // BE,),
        in_specs=[
            pl.BlockSpec((BE, DE2), lambda i: (i, 0)),
            pl.BlockSpec((DE2, D), lambda i: (0, 0)),
            pl.BlockSpec((DE2, D), lambda i: (0, 0)),
            pl.BlockSpec((1, D), lambda i: (0, 0)),
            pl.BlockSpec((1, D), lambda i: (0, 0)),
        ],
        out_specs=pl.BlockSpec((BE, D), lambda i: (i, 0)),
        out_shape=jax.ShapeDtypeStruct((Ep2, D), jnp.int32),
        compiler_params=pltpu.CompilerParams(
            dimension_semantics=("parallel",)
        ),
    )(ea2, wa2, wb2, ba_row, bb_row)


def _sc_agg(x, src, dst, e, zblk, n_chunks, n_pad):
    N, D = x.shape
    assert n_chunks % 2 == 1

    def body(x_hbm, src_hbm, dst_hbm, e_hbm, z_hbm, out_hbm,
             idx_s0, idx_d0, xj0, ev0, idx_s1, idx_d1, xj1, ev1,
             sg0, se0, sg1, se1, agg_sh):
        cid = lax.axis_index("c")
        sid = lax.axis_index("s")
        wid = sid * NC + cid
        zrows = n_pad // NS
        pltpu.sync_copy(z_hbm, agg_sh.at[pl.ds(sid * zrows, zrows)])
        plsc.subcore_barrier()

        base = wid * (n_chunks * K)
        bufs = ((idx_s0, idx_d0, xj0, ev0, sg0, se0),
                (idx_s1, idx_d1, xj1, ev1, sg1, se1))

        def start(j, b):
            idx_s, idx_d, xj, ev, sg, se = bufs[b]
            off = base + j * K
            pltpu.sync_copy(src_hbm.at[pl.ds(off, K)], idx_s)
            pltpu.sync_copy(dst_hbm.at[pl.ds(off, K)], idx_d)
            pltpu.async_copy(x_hbm.at[idx_s], xj, sg)
            pltpu.async_copy(e_hbm.at[pl.ds(pl.multiple_of(off // 2, 8), K // 2), :], ev, se)

        def finish(j, b):
            idx_s, idx_d, xj, ev, sg, se = bufs[b]
            off = base + j * K
            pltpu.make_async_copy(x_hbm.at[idx_s], xj, sg).wait()
            pltpu.make_async_copy(e_hbm.at[pl.ds(pl.multiple_of(off // 2, 8), K // 2), :],
                                  ev, se).wait()

            @plsc.parallel_loop(0, K // 2, unroll=2)
            def _pair(q):
                for half in range(2):
                    r = 2 * q + half
                    for g in range(D // (2 * LANES)):
                        eb = plsc.bitcast(
                            ev[q, pl.ds(64 * half + LANES * g, LANES)],
                            jnp.bfloat16)
                        a, b = plsc.unpack(
                            eb, format=plsc.PackFormat.INTERLEAVED)
                        ca = pl.ds(2 * LANES * g, LANES)
                        cb = pl.ds(2 * LANES * g + LANES, LANES)
                        xj[r, ca] = jnp.maximum(xj[r, ca] + a, 0.0)
                        xj[r, cb] = jnp.maximum(xj[r, cb] + b, 0.0)

            pltpu.sync_copy(xj, agg_sh.at[idx_d], add=True)

        start(0, 0)

        @pl.loop(0, n_chunks - 1, step=2)
        def _chunk(t):
            start(t + 1, 1)
            finish(t, 0)
            start(t + 2, 0)
            finish(t + 1, 1)

        finish(n_chunks - 1, 0)

        plsc.subcore_barrier()
        drows = ((N // NS) + 7) // 8 * 8
        last = N - (NS - 1) * drows

        @pl.when(sid < NS - 1)
        def _():
            pltpu.sync_copy(
                agg_sh.at[pl.ds(sid * drows, drows)],
                out_hbm.at[cid, pl.ds(sid * drows, drows), :],
            )

        @pl.when(sid == NS - 1)
        def _():
            pltpu.sync_copy(
                agg_sh.at[pl.ds((NS - 1) * drows, last)],
                out_hbm.at[cid, pl.ds((NS - 1) * drows, last), :],
            )

    mesh = plsc.VectorSubcoreMesh(core_axis_name="c", subcore_axis_name="s")
    kern = pl.kernel(
        body,
        out_type=jax.ShapeDtypeStruct((NC, N, D), jnp.float32),
        mesh=mesh,
        compiler_params=pltpu.CompilerParams(needs_layout_passes=False),
        scratch_types=[
            pltpu.VMEM((K,), jnp.int32),
            pltpu.VMEM((K,), jnp.int32),
            pltpu.VMEM((K, D), jnp.float32),
            pltpu.VMEM((K // 2, D), jnp.int32),
            pltpu.VMEM((K,), jnp.int32),
            pltpu.VMEM((K,), jnp.int32),
            pltpu.VMEM((K, D), jnp.float32),
            pltpu.VMEM((K // 2, D), jnp.int32),
            pltpu.SemaphoreType.DMA,
            pltpu.SemaphoreType.DMA,
            pltpu.SemaphoreType.DMA,
            pltpu.SemaphoreType.DMA,
            pltpu.VMEM_SHARED((n_pad, D), jnp.float32),
        ],
    )
    return kern(x, src, dst, e, zblk)


def _node_mlp_body(x_ref, p_ref, w_ref, b_ref, o_ref):
    s = x_ref[...] + p_ref[0] + p_ref[1]
    h = jnp.dot(s, w_ref[...], preferred_element_type=jnp.float32) + b_ref[...]
    o_ref[...] = jnp.maximum(h, 0.0)


def _node_mlp(x, partials, w_t, b_row):
    N, D = x.shape
    BN = 1000
    assert N % BN == 0
    return pl.pallas_call(
        _node_mlp_body,
        grid=(N // BN,),
        in_specs=[
            pl.BlockSpec((BN, D), lambda i: (i, 0)),
            pl.BlockSpec((NC, BN, D), lambda i: (0, i, 0)),
            pl.BlockSpec((D, D), lambda i: (0, 0)),
            pl.BlockSpec((1, D), lambda i: (0, 0)),
        ],
        out_specs=pl.BlockSpec((BN, D), lambda i: (i, 0)),
        out_shape=jax.ShapeDtypeStruct((N, D), jnp.float32),
        compiler_params=pltpu.CompilerParams(
            dimension_semantics=("parallel",)
        ),
    )(x, partials, w_t, b_row)


def kernel(x, edge_index, edge_attr, W_edge, b_edge, W_mlp, b_mlp):
    N, D = x.shape
    E = edge_index.shape[1]
    DE = edge_attr.shape[1]

    per_round = TILES * K
    assert E % per_round == 0
    n_chunks = E // per_round
    assert n_chunks % 2 == 1
    Ep = E

    src = edge_index[0]
    dst = edge_index[1]
    ea = edge_attr

    n_pad = ((N + 1 + 8 * NS - 1) // (8 * NS)) * (8 * NS)
    zblk = jnp.zeros((n_pad // NS, D), jnp.float32)

    w = np.arange(D // 2)
    cols_a = 32 * (w // 16) + (w % 16)
    cols_b = cols_a + 16
    wt = W_edge.T

    def blockdiag(m):
        z = jnp.zeros_like(m)
        return jnp.concatenate([jnp.concatenate([m, z], axis=1),
                                jnp.concatenate([z, m], axis=1)], axis=0)

    ea2 = ea.reshape(E // 2, 2 * DE)
    e32 = _edge_proj(ea2, blockdiag(wt[:, cols_a]), blockdiag(wt[:, cols_b]),
                     jnp.tile(b_edge[cols_a], 2)[None, :],
                     jnp.tile(b_edge[cols_b], 2)[None, :])
    partials = _sc_agg(x, src, dst, e32, zblk, n_chunks, n_pad)
    return _node_mlp(x, partials, W_mlp.T, b_mlp[None, :])


# speedup vs baseline: 1.0031x; 1.0031x over previous
import functools

import jax
import jax.numpy as jnp
import numpy as np
from jax import lax
from jax.experimental import pallas as pl
from jax.experimental.pallas import tpu as pltpu
from jax.experimental.pallas import tpu_sc as plsc

NC, NS, LANES = 2, 16, 16
TILES = NC * NS
K = 80


def _edge_proj_body(ea_ref, wa_ref, wb_ref, ba_ref, bb_ref, o_ref):
    a = (jnp.dot(ea_ref[...], wa_ref[...], preferred_element_type=jnp.float32)
         + ba_ref[...])
    b = (jnp.dot(ea_ref[...], wb_ref[...], preferred_element_type=jnp.float32)
         + bb_ref[...])
    au = jax.lax.bitcast_convert_type(
        a.astype(jnp.bfloat16), jnp.uint16).astype(jnp.uint32)
    bu = jax.lax.bitcast_convert_type(
        b.astype(jnp.bfloat16), jnp.uint16).astype(jnp.uint32)
    o_ref[...] = jax.lax.bitcast_convert_type(au | (bu << 16), jnp.int32)


def _edge_proj(ea, wa, wb, ba_row, bb_row):
    Ep, DE = ea.shape
    H = wa.shape[1]
    BE = 5 * TILES * K
    assert Ep % BE == 0
    return pl.pallas_call(
        _edge_proj_body,
        grid=(Ep // BE,),
        in_specs=[
            pl.BlockSpec((BE, DE), lambda i: (i, 0)),
            pl.BlockSpec((DE, H), lambda i: (0, 0)),
            pl.BlockSpec((DE, H), lambda i: (0, 0)),
            pl.BlockSpec((1, H), lambda i: (0, 0)),
            pl.BlockSpec((1, H), lambda i: (0, 0)),
        ],
        out_specs=pl.BlockSpec((BE, H), lambda i: (i, 0)),
        out_shape=jax.ShapeDtypeStruct((Ep, H), jnp.int32),
        compiler_params=pltpu.CompilerParams(
            dimension_semantics=("parallel",)
        ),
    )(ea, wa, wb, ba_row, bb_row)


def _sc_agg(x, src, dst, e, zblk, n_chunks, n_pad):
    N, D = x.shape
    assert n_chunks % 2 == 1

    def body(x_hbm, src_hbm, dst_hbm, e_hbm, z_hbm, out_hbm,
             idx_s0, idx_d0, xj0, ev0, idx_s1, idx_d1, xj1, ev1,
             sg0, se0, sg1, se1, agg_sh):
        cid = lax.axis_index("c")
        sid = lax.axis_index("s")
        wid = sid * NC + cid
        zrows = n_pad // NS
        pltpu.sync_copy(z_hbm, agg_sh.at[pl.ds(sid * zrows, zrows)])
        plsc.subcore_barrier()

        base = wid * (n_chunks * K)
        bufs = ((idx_s0, idx_d0, xj0, ev0, sg0, se0),
                (idx_s1, idx_d1, xj1, ev1, sg1, se1))

        def start(j, b):
            idx_s, idx_d, xj, ev, sg, se = bufs[b]
            off = base + j * K
            pltpu.sync_copy(src_hbm.at[pl.ds(off, K)], idx_s)
            pltpu.sync_copy(dst_hbm.at[pl.ds(off, K)], idx_d)
            pltpu.async_copy(x_hbm.at[idx_s], xj, sg)
            pltpu.async_copy(e_hbm.at[pl.ds(off, K), :], ev, se)

        def finish(j, b):
            idx_s, idx_d, xj, ev, sg, se = bufs[b]
            off = base + j * K
            pltpu.make_async_copy(x_hbm.at[idx_s], xj, sg).wait()
            pltpu.make_async_copy(e_hbm.at[pl.ds(off, K), :], ev, se).wait()

            @plsc.parallel_loop(0, K, unroll=2)
            def _row(r):
                for g in range(D // (2 * LANES)):
                    eb = plsc.bitcast(ev[r, pl.ds(LANES * g, LANES)],
                                      jnp.bfloat16)
                    a, b = plsc.unpack(eb, format=plsc.PackFormat.INTERLEAVED)
                    ca = pl.ds(2 * LANES * g, LANES)
                    cb = pl.ds(2 * LANES * g + LANES, LANES)
                    xj[r, ca] = jnp.maximum(xj[r, ca] + a, 0.0)
                    xj[r, cb] = jnp.maximum(xj[r, cb] + b, 0.0)

            pltpu.sync_copy(xj, agg_sh.at[idx_d], add=True)

        start(0, 0)

        @pl.loop(0, n_chunks - 1, step=2)
        def _chunk(t):
            start(t + 1, 1)
            finish(t, 0)
            start(t + 2, 0)
            finish(t + 1, 1)

        finish(n_chunks - 1, 0)

        plsc.subcore_barrier()
        drows = ((N // NS) + 7) // 8 * 8
        last = N - (NS - 1) * drows

        @pl.when(sid < NS - 1)
        def _():
            pltpu.sync_copy(
                agg_sh.at[pl.ds(sid * drows, drows)],
                out_hbm.at[cid, pl.ds(sid * drows, drows), :],
            )

        @pl.when(sid == NS - 1)
        def _():
            pltpu.sync_copy(
                agg_sh.at[pl.ds((NS - 1) * drows, last)],
                out_hbm.at[cid, pl.ds((NS - 1) * drows, last), :],
            )

    mesh = plsc.VectorSubcoreMesh(core_axis_name="c", subcore_axis_name="s")
    kern = pl.kernel(
        body,
        out_type=jax.ShapeDtypeStruct((NC, N, D), jnp.float32),
        mesh=mesh,
        compiler_params=pltpu.CompilerParams(needs_layout_passes=False),
        scratch_types=[
            pltpu.VMEM((K,), jnp.int32),
            pltpu.VMEM((K,), jnp.int32),
            pltpu.VMEM((K, D), jnp.float32),
            pltpu.VMEM((K, D // 2), jnp.int32),
            pltpu.VMEM((K,), jnp.int32),
            pltpu.VMEM((K,), jnp.int32),
            pltpu.VMEM((K, D), jnp.float32),
            pltpu.VMEM((K, D // 2), jnp.int32),
            pltpu.SemaphoreType.DMA,
            pltpu.SemaphoreType.DMA,
            pltpu.SemaphoreType.DMA,
            pltpu.SemaphoreType.DMA,
            pltpu.VMEM_SHARED((n_pad, D), jnp.float32),
        ],
    )
    return kern(x, src, dst, e, zblk)


def _node_mlp_body(x_ref, p_ref, w_ref, b_ref, o_ref):
    s = x_ref[...] + p_ref[0] + p_ref[1]
    h = jnp.dot(s, w_ref[...], preferred_element_type=jnp.float32) + b_ref[...]
    o_ref[...] = jnp.maximum(h, 0.0)


def _node_mlp(x, partials, w_t, b_row):
    N, D = x.shape
    BN = 1000
    assert N % BN == 0
    return pl.pallas_call(
        _node_mlp_body,
        grid=(N // BN,),
        in_specs=[
            pl.BlockSpec((BN, D), lambda i: (i, 0)),
            pl.BlockSpec((NC, BN, D), lambda i: (0, i, 0)),
            pl.BlockSpec((D, D), lambda i: (0, 0)),
            pl.BlockSpec((1, D), lambda i: (0, 0)),
        ],
        out_specs=pl.BlockSpec((BN, D), lambda i: (i, 0)),
        out_shape=jax.ShapeDtypeStruct((N, D), jnp.float32),
        compiler_params=pltpu.CompilerParams(
            dimension_semantics=("parallel",)
        ),
    )(x, partials, w_t, b_row)


def kernel(x, edge_index, edge_attr, W_edge, b_edge, W_mlp, b_mlp):
    N, D = x.shape
    E = edge_index.shape[1]
    DE = edge_attr.shape[1]

    per_round = TILES * K
    assert E % per_round == 0
    n_chunks = E // per_round
    assert n_chunks % 2 == 1
    Ep = E

    src = edge_index[0]
    dst = edge_index[1]
    ea = edge_attr

    n_pad = ((N + 1 + 8 * NS - 1) // (8 * NS)) * (8 * NS)
    zblk = jnp.zeros((n_pad // NS, D), jnp.float32)

    w = np.arange(D // 2)
    cols_a = 32 * (w // 16) + (w % 16)
    cols_b = cols_a + 16
    wt = W_edge.T

    e32 = _edge_proj(ea, wt[:, cols_a], wt[:, cols_b],
                     b_edge[cols_a][None, :], b_edge[cols_b][None, :])
    partials = _sc_agg(x, src, dst, e32, zblk, n_chunks, n_pad)
    return _node_mlp(x, partials, W_mlp.T, b_mlp[None, :])
